# baseline (speedup 1.0000x reference)
import functools

import jax
import jax.numpy as jnp
import numpy as np
from jax import lax
from jax.experimental import pallas as pl
from jax.experimental.pallas import tpu as pltpu
from jax.experimental.pallas import tpu_sc as plsc

_N_OUT = 50
_TEMPLATE = 10
_FDIM = 512
_N_PAIRS = (_N_OUT * (_N_OUT - 1)) // 2
_N_PAIRS_PAD = 1232
_LANES = 16

_IU, _JU = np.triu_indices(_N_OUT, k=1)
_IUP = np.zeros((_N_PAIRS_PAD, 1), dtype=np.int32)
_JUP = np.zeros((_N_PAIRS_PAD, 1), dtype=np.int32)
_IUP[:_N_PAIRS, 0] = _IU
_JUP[:_N_PAIRS, 0] = _JU


def _sc_gather_body(cf_hbm, sel_hbm, rows_hbm, idx_v, rows_v, sem):
    pltpu.sync_copy(sel_hbm.at[pl.ds(0, _LANES)], idx_v)
    pltpu.async_copy(cf_hbm.at[idx_v], rows_v, sem).wait()
    pltpu.sync_copy(rows_v, rows_hbm)


_sc_gather = functools.partial(
    pl.kernel,
    out_type=jax.ShapeDtypeStruct((_LANES, _FDIM), jnp.float32),
    mesh=plsc.VectorSubcoreMesh(core_axis_name="c", subcore_axis_name="s",
                                num_cores=1, num_subcores=1),
    scratch_types=[
        pltpu.VMEM((_LANES,), jnp.int32),
        pltpu.VMEM((_LANES, _FDIM), jnp.float32),
        pltpu.SemaphoreType.DMA,
    ],
)(_sc_gather_body)


def _tc_body(rows_ref, noise_ref, W1_ref, b1_ref, W2_ref, b2_ref,
             W3_ref, b3_ref, We1_ref, be1_ref, We2r_ref, be2_ref,
             iu_ref, ju_ref, tf_out_ref, ep_out_ref):
    p = jnp.sum(rows_ref[0:_TEMPLATE, :], axis=0,
                keepdims=True) * (1.0 / _TEMPLATE)
    h1 = jnp.maximum(
        jnp.dot(p, W1_ref[...], preferred_element_type=jnp.float32)
        + b1_ref[...], 0.0)
    h2 = jnp.maximum(
        jnp.dot(h1, W2_ref[...], preferred_element_type=jnp.float32)
        + b2_ref[...], 0.0)
    h3 = jax.nn.sigmoid(
        jnp.dot(h2, W3_ref[...], preferred_element_type=jnp.float32)
        + b3_ref[...])

    base = jnp.broadcast_to(h3, (_N_OUT, _FDIM))
    noise_full = jnp.concatenate(
        [jnp.zeros((_TEMPLATE, _FDIM), jnp.float32), noise_ref[...]],
        axis=0)
    tf = base + 0.1 * noise_full
    tf_out_ref[...] = tf

    A = jnp.dot(tf, We1_ref[0:_FDIM, :],
                preferred_element_type=jnp.float32)
    B = jnp.dot(tf, We1_ref[_FDIM:2 * _FDIM, :],
                preferred_element_type=jnp.float32)
    cols = lax.broadcasted_iota(jnp.int32, (_N_PAIRS_PAD, _N_OUT), 1)
    ohi = (cols == iu_ref[...]).astype(jnp.float32)
    ohj = (cols == ju_ref[...]).astype(jnp.float32)
    Ai = jnp.dot(ohi, A, preferred_element_type=jnp.float32)
    Bj = jnp.dot(ohj, B, preferred_element_type=jnp.float32)
    e = jnp.maximum(Ai + Bj + be1_ref[...], 0.0)
    s = jnp.sum(e * We2r_ref[...], axis=1, keepdims=True) + be2_ref[...]
    ep_out_ref[...] = jax.nn.sigmoid(s)


def kernel(clean_features, selected_nodes, noise, W1, b1, W2, b2, W3, b3,
           We1, be1, We2, be2):
    rows = _sc_gather(clean_features, selected_nodes)
    tf, ep = pl.pallas_call(
        _tc_body,
        out_shape=[
            jax.ShapeDtypeStruct((_N_OUT, _FDIM), jnp.float32),
            jax.ShapeDtypeStruct((_N_PAIRS_PAD, 1), jnp.float32),
        ],
    )(rows, noise,
      W1, b1.reshape(1, 64), W2, b2.reshape(1, 64), W3, b3.reshape(1, _FDIM),
      We1, be1.reshape(1, 64), We2.reshape(1, 64), be2.reshape(1, 1),
      jnp.asarray(_IUP), jnp.asarray(_JUP))
    return (tf, ep[:_N_PAIRS])


# speedup vs baseline: 1.0103x; 1.0103x over previous
import functools

import jax
import jax.numpy as jnp
import numpy as np
from jax import lax
from jax.experimental import pallas as pl
from jax.experimental.pallas import tpu as pltpu
from jax.experimental.pallas import tpu_sc as plsc

_N_OUT = 50
_TEMPLATE = 10
_FDIM = 512
_N_PAIRS = (_N_OUT * (_N_OUT - 1)) // 2
_N_PAIRS_PAD = 1232
_LANES = 16

_IU, _JU = np.triu_indices(_N_OUT, k=1)
_IUP = np.zeros((_N_PAIRS_PAD, 1), dtype=np.int32)
_JUP = np.zeros((_N_PAIRS_PAD, 1), dtype=np.int32)
_IUP[:_N_PAIRS, 0] = _IU
_JUP[:_N_PAIRS, 0] = _JU


def _sc_gather_body(cf_hbm, sel_hbm, rows_hbm, idx_v, rows_v, sem):
    pltpu.sync_copy(sel_hbm.at[pl.ds(0, _LANES)], idx_v)
    pltpu.async_copy(cf_hbm.at[idx_v], rows_v, sem).wait()
    pltpu.sync_copy(rows_v, rows_hbm)


_sc_gather = functools.partial(
    pl.kernel,
    out_type=jax.ShapeDtypeStruct((_LANES, _FDIM), jnp.float32),
    mesh=plsc.VectorSubcoreMesh(core_axis_name="c", subcore_axis_name="s",
                                num_cores=1, num_subcores=1),
    scratch_types=[
        pltpu.VMEM((_LANES,), jnp.int32),
        pltpu.VMEM((_LANES, _FDIM), jnp.float32),
        pltpu.SemaphoreType.DMA,
    ],
)(_sc_gather_body)


def _tc_noise_body(noise_ref, We1_ref, iu_ref, ju_ref, g_out_ref):
    nsc = 0.1 * noise_ref[...]
    NA = jnp.dot(nsc, We1_ref[0:_FDIM, :],
                 preferred_element_type=jnp.float32)
    NB = jnp.dot(nsc, We1_ref[_FDIM:2 * _FDIM, :],
                 preferred_element_type=jnp.float32)
    zeros = jnp.zeros((_TEMPLATE, 64), jnp.float32)
    NAf = jnp.concatenate([zeros, NA], axis=0)
    NBf = jnp.concatenate([zeros, NB], axis=0)
    cols = lax.broadcasted_iota(jnp.int32, (_N_PAIRS_PAD, _N_OUT), 1)
    ohi = (cols == iu_ref[...]).astype(jnp.float32)
    ohj = (cols == ju_ref[...]).astype(jnp.float32)
    g_out_ref[...] = (
        jnp.dot(ohi, NAf, preferred_element_type=jnp.float32)
        + jnp.dot(ohj, NBf, preferred_element_type=jnp.float32))


def _tc_main_body(rows_ref, noise_ref, W1_ref, b1_ref, W2_ref, b2_ref,
                  W3_ref, b3_ref, We1_ref, be1_ref, We2r_ref, be2_ref,
                  g_ref, tf_out_ref, ep_out_ref):
    p = jnp.sum(rows_ref[0:_TEMPLATE, :], axis=0,
                keepdims=True) * (1.0 / _TEMPLATE)
    h1 = jnp.maximum(
        jnp.dot(p, W1_ref[...], preferred_element_type=jnp.float32)
        + b1_ref[...], 0.0)
    h2 = jnp.maximum(
        jnp.dot(h1, W2_ref[...], preferred_element_type=jnp.float32)
        + b2_ref[...], 0.0)
    h3 = jax.nn.sigmoid(
        jnp.dot(h2, W3_ref[...], preferred_element_type=jnp.float32)
        + b3_ref[...])

    base = jnp.broadcast_to(h3, (_N_OUT, _FDIM))
    noise_full = jnp.concatenate(
        [jnp.zeros((_TEMPLATE, _FDIM), jnp.float32), noise_ref[...]],
        axis=0)
    tf = base + 0.1 * noise_full
    tf_out_ref[...] = tf

    a = jnp.dot(h3, We1_ref[0:_FDIM, :],
                preferred_element_type=jnp.float32)
    b = jnp.dot(h3, We1_ref[_FDIM:2 * _FDIM, :],
                preferred_element_type=jnp.float32)
    c = a + b + be1_ref[...]
    e = jnp.maximum(g_ref[...] + c, 0.0)
    s = jnp.sum(e * We2r_ref[...], axis=1, keepdims=True) + be2_ref[...]
    ep_out_ref[...] = jax.nn.sigmoid(s)


def kernel(clean_features, selected_nodes, noise, W1, b1, W2, b2, W3, b3,
           We1, be1, We2, be2):
    rows = _sc_gather(clean_features, selected_nodes)
    g = pl.pallas_call(
        _tc_noise_body,
        out_shape=jax.ShapeDtypeStruct((_N_PAIRS_PAD, 64), jnp.float32),
    )(noise, We1, jnp.asarray(_IUP), jnp.asarray(_JUP))
    tf, ep = pl.pallas_call(
        _tc_main_body,
        out_shape=[
            jax.ShapeDtypeStruct((_N_OUT, _FDIM), jnp.float32),
            jax.ShapeDtypeStruct((_N_PAIRS_PAD, 1), jnp.float32),
        ],
    )(rows, noise,
      W1, b1.reshape(1, 64), W2, b2.reshape(1, 64), W3, b3.reshape(1, _FDIM),
      We1, be1.reshape(1, 64), We2.reshape(1, 64), be2.reshape(1, 1),
      g)
    return (tf, ep[:_N_PAIRS])
